# Initial kernel scaffold; baseline (speedup 1.0000x reference)
import functools

import jax
import jax.numpy as jnp
from jax import lax
from jax.experimental import pallas as pl
from jax.experimental.pallas import tpu as pltpu
from jax.experimental.pallas import tpu_sc as plsc

VOCAB = 1000
EMBED = 64
B = 4096
L = 200
NC = 2
NS = 16
NW = NC * NS
ROWS_PER_W = B // NW
CH = 2
NCHUNK = ROWS_PER_W // CH
SUB = 100
NSUB = (CH * L) // SUB
XROWS_PER_W = ROWS_PER_W * L // SUB


def _sc_kernel():
    mesh = plsc.VectorSubcoreMesh(core_axis_name="c", subcore_axis_name="s")

    @functools.partial(
        pl.kernel,
        mesh=mesh,
        out_type=jax.ShapeDtypeStruct((B, L, EMBED), jnp.float32),
        scratch_types=[
            pltpu.VMEM((XROWS_PER_W, SUB), jnp.int32),
            pltpu.VMEM((L, EMBED), jnp.float32),
            pltpu.VMEM((CH, L, EMBED), jnp.float32),
            pltpu.SemaphoreType.DMA,
        ],
    )
    def k(x_hbm, ww_hbm, wp_hbm, out_hbm, x_v, pos_v, buf, gsem):
        wid = lax.axis_index("s") * NC + lax.axis_index("c")
        base_row = wid * ROWS_PER_W
        pltpu.sync_copy(wp_hbm.at[pl.ds(0, L)], pos_v)
        pltpu.sync_copy(x_hbm.at[pl.ds(wid * XROWS_PER_W, XROWS_PER_W)], x_v)

        def chunk_body(g, carry):
            handles = []
            for j in range(NSUB):
                r, off = divmod(j * SUB, L)
                handles.append(
                    pltpu.async_copy(
                        ww_hbm.at[x_v.at[g * NSUB + j]],
                        buf.at[r, pl.ds(off, SUB)],
                        gsem,
                    )
                )
            for h in handles:
                h.wait()

            def pos_body(p, c2):
                for kk in range(EMBED // 16):
                    pv = pos_v[p, pl.ds(kk * 16, 16)]
                    for r in range(CH):
                        plsc.addupdate(buf.at[r, p, pl.ds(kk * 16, 16)], pv)
                return c2

            lax.fori_loop(0, L, pos_body, 0)
            pltpu.sync_copy(buf, out_hbm.at[pl.ds(base_row + g * CH, CH)])
            return carry

        lax.fori_loop(0, NCHUNK, chunk_body, 0)

    return k


_call = _sc_kernel()


@jax.jit
def kernel(x, W_words, W_pos):
    x2 = x.reshape(B * L // SUB, SUB).astype(jnp.int32)
    return _call(x2, W_words, W_pos)


# speedup vs baseline: 3.4932x; 3.4932x over previous
import functools

import jax
import jax.numpy as jnp
from jax import lax
from jax.experimental import pallas as pl
from jax.experimental.pallas import tpu as pltpu
from jax.experimental.pallas import tpu_sc as plsc

VOCAB = 1000
EMBED = 64
B = 4096
L = 200
NC = 2
NS = 16
NW = NC * NS
ROWS_PER_W = B // NW
CH = 2
NCHUNK = ROWS_PER_W // CH
SUB = 100
NSUB = (CH * L) // SUB
XROWS_PER_W = ROWS_PER_W * L // SUB


@functools.cache
def _sc_kernel():
    mesh = plsc.VectorSubcoreMesh(core_axis_name="c", subcore_axis_name="s")

    @functools.partial(
        pl.kernel,
        mesh=mesh,
        out_type=jax.ShapeDtypeStruct((B, L, EMBED), jnp.float32),
        compiler_params=pltpu.CompilerParams(use_tc_tiling_on_sc=False),
        scratch_types=[
            pltpu.VMEM((XROWS_PER_W, SUB), jnp.int32),
            pltpu.VMEM((L, EMBED), jnp.float32),
            pltpu.VMEM((CH, L, EMBED), jnp.float32),
            pltpu.SemaphoreType.DMA,
        ],
    )
    def k(x_hbm, ww_hbm, wp_hbm, out_hbm, x_v, pos_v, buf, gsem):
        wid = lax.axis_index("s") * NC + lax.axis_index("c")
        base_row = wid * ROWS_PER_W
        pltpu.sync_copy(wp_hbm.at[pl.ds(0, L)], pos_v)
        pltpu.sync_copy(x_hbm.at[pl.ds(wid * XROWS_PER_W, XROWS_PER_W)], x_v)

        def chunk_body(g, carry):
            handles = []
            for j in range(NSUB):
                r, off = divmod(j * SUB, L)
                handles.append(
                    pltpu.async_copy(
                        ww_hbm.at[x_v.at[g * NSUB + j]],
                        buf.at[r, pl.ds(off, SUB)],
                        gsem,
                    )
                )
            for h in handles:
                h.wait()

            def pos_body(p, c2):
                for kk in range(EMBED // 16):
                    pv = pos_v[p, pl.ds(kk * 16, 16)]
                    for r in range(CH):
                        plsc.addupdate(buf.at[r, p, pl.ds(kk * 16, 16)], pv)
                return c2

            lax.fori_loop(0, L, pos_body, 0)
            pltpu.sync_copy(buf, out_hbm.at[pl.ds(base_row + g * CH, CH)])
            return carry

        lax.fori_loop(0, NCHUNK, chunk_body, 0)

    return k


@jax.jit
def kernel(x, W_words, W_pos):
    x2 = x.reshape(B * L // SUB, SUB).astype(jnp.int32)
    return _sc_kernel()(x2, W_words, W_pos)


# speedup vs baseline: 3.5424x; 1.0141x over previous
import functools

import jax
import jax.numpy as jnp
from jax import lax
from jax.experimental import pallas as pl
from jax.experimental.pallas import tpu as pltpu
from jax.experimental.pallas import tpu_sc as plsc

VOCAB = 1000
EMBED = 64
B = 4096
L = 200
NC = 2
NS = 16
NW = NC * NS
ROWS_PER_W = B // NW
CH = 2
NCHUNK = ROWS_PER_W // CH
SUB = 100
NSUB = (CH * L) // SUB
XROWS_PER_W = ROWS_PER_W * L // SUB


@functools.cache
def _sc_kernel():
    mesh = plsc.VectorSubcoreMesh(core_axis_name="c", subcore_axis_name="s")

    @functools.partial(
        pl.kernel,
        mesh=mesh,
        out_type=jax.ShapeDtypeStruct((B, L, EMBED), jnp.float32),
        compiler_params=pltpu.CompilerParams(use_tc_tiling_on_sc=False),
        scratch_types=[
            pltpu.VMEM((XROWS_PER_W, SUB), jnp.int32),
            pltpu.VMEM((L, EMBED), jnp.float32),
            pltpu.VMEM((CH, L, EMBED), jnp.float32),
            pltpu.VMEM((CH, L, EMBED), jnp.float32),
            pltpu.SemaphoreType.DMA,
            pltpu.SemaphoreType.DMA,
            pltpu.SemaphoreType.DMA,
            pltpu.SemaphoreType.DMA,
        ],
    )
    def k(x_hbm, ww_hbm, wp_hbm, out_hbm,
          x_v, pos_v, buf0, buf1, gsem0, gsem1, wsem0, wsem1):
        wid = lax.axis_index("s") * NC + lax.axis_index("c")
        base_row = wid * ROWS_PER_W
        pltpu.sync_copy(wp_hbm.at[pl.ds(0, L)], pos_v)
        pltpu.sync_copy(x_hbm.at[pl.ds(wid * XROWS_PER_W, XROWS_PER_W)], x_v)

        def issue_gathers(c, buf, gsem):
            for j in range(NSUB):
                r, off = divmod(j * SUB, L)
                pltpu.async_copy(
                    ww_hbm.at[x_v.at[c * NSUB + j]],
                    buf.at[r, pl.ds(off, SUB)],
                    gsem,
                )

        def drain(sem, buf):
            pltpu.make_async_copy(out_hbm.at[pl.ds(0, CH)], buf, sem).wait()

        def add_pos(buf):
            def pos_body(p, c2):
                for kk in range(EMBED // 16):
                    pv = pos_v[p, pl.ds(kk * 16, 16)]
                    for r in range(CH):
                        plsc.addupdate(buf.at[r, p, pl.ds(kk * 16, 16)], pv)
                return c2

            lax.fori_loop(0, L, pos_body, 0)

        issue_gathers(0, buf0, gsem0)

        def pair_body(m, carry):
            @pl.when(m >= 1)
            def _():
                drain(wsem1, buf1)

            issue_gathers(2 * m + 1, buf1, gsem1)
            drain(gsem0, buf0)
            add_pos(buf0)
            pltpu.async_copy(
                buf0, out_hbm.at[pl.ds(base_row + (2 * m) * CH, CH)], wsem0)

            @pl.when(m < NCHUNK // 2 - 1)
            def _():
                drain(wsem0, buf0)
                issue_gathers(2 * m + 2, buf0, gsem0)

            drain(gsem1, buf1)
            add_pos(buf1)
            pltpu.async_copy(
                buf1, out_hbm.at[pl.ds(base_row + (2 * m + 1) * CH, CH)], wsem1)
            return carry

        lax.fori_loop(0, NCHUNK // 2, pair_body, 0)
        drain(wsem0, buf0)
        drain(wsem1, buf1)

    return k


@jax.jit
def kernel(x, W_words, W_pos):
    x2 = x.reshape(B * L // SUB, SUB).astype(jnp.int32)
    return _sc_kernel()(x2, W_words, W_pos)
